# baseline (device time: 27544 ns/iter reference)
import jax
import jax.numpy as jnp
from jax import lax
from jax.experimental import pallas as pl
from jax.experimental.pallas import tpu as pltpu

N_DEV = 4
GRAY = [0, 1, 1, 0]


def _gelu(y):
    c = 0.7978845608028654
    return 0.5 * y * (1.0 + jnp.tanh(c * (y + 0.044715 * (y * y * y))))


def kernel(x, w_mat):
    m, _ = x.shape
    _, n = w_mat.shape
    half_r = m // 2
    q_r = m // 4
    e_r = m // 8
    half_c = n // 2

    def body(xh_ref, wh_ref, out_ref,
             x_ref, w_ref, ob, sb1, rb1, sb2, rb2, sb3, rb3, rb4,
             send_sems, recv_sems, out_sems, in_sems):
        my = lax.axis_index("i")
        p1t = lax.bitwise_xor(my, 1)
        p2t = 3 - my

        barrier = pltpu.get_barrier_semaphore()
        pl.semaphore_signal(barrier, inc=1, device_id=(p1t,),
                            device_id_type=pl.DeviceIdType.MESH)
        pl.semaphore_signal(barrier, inc=1, device_id=(p2t,),
                            device_id_type=pl.DeviceIdType.MESH)

        cp_w = pltpu.make_async_copy(wh_ref, w_ref, in_sems.at[0])
        cp_w.start()
        cp_x = pltpu.make_async_copy(xh_ref, x_ref, in_sems.at[1])
        cp_x.start()

        cols = [slice(0, half_c), slice(half_c, n)]
        cp_w.wait()
        wb = [w_ref[:, cols[i]].astype(jnp.bfloat16) for i in range(2)]
        cp_x.wait()

        def run(dev):
            p1, p2 = dev ^ 1, 3 - dev
            hh = [GRAY[dev], dev >> 1]
            qq = [dev >> 1, dev & 1]
            s1_partner = [p1, p2]
            s2_partner = [p2, p1]
            KH = [hh[i] * half_r for i in range(2)]
            SH = [(1 - hh[i]) * half_r for i in range(2)]

            odmas = []

            def store_out(r0, inst, value):
                ob[r0:r0 + e_r, cols[inst]] = value
                cp = pltpu.make_async_copy(
                    ob.at[r0:r0 + e_r, cols[inst]],
                    out_ref.at[r0:r0 + e_r, cols[inst]],
                    out_sems.at[len(odmas)])
                cp.start()
                odmas.append(cp)

            def dot_rows(r0, nrows, inst):
                xb = x_ref[r0:r0 + nrows, :].astype(jnp.bfloat16)
                return jnp.dot(xb, wb[inst], preferred_element_type=jnp.float32)

            def mk(src, dst, idx, tgt):
                return pltpu.make_async_remote_copy(
                    src_ref=src, dst_ref=dst,
                    send_sem=send_sems.at[idx], recv_sem=recv_sems.at[idx],
                    device_id=(tgt,), device_id_type=pl.DeviceIdType.MESH)

            jf = [1 - qq[0], qq[1]]
            s1d = [[None, None], [None, None]]
            first = True
            for inst, j in [(0, jf[0]), (1, jf[1]),
                            (0, 1 - jf[0]), (1, 1 - jf[1])]:
                sb1[inst, j] = dot_rows(SH[inst] + j * q_r, q_r,
                                        inst).astype(jnp.bfloat16)
                if first:
                    pl.semaphore_wait(barrier, 2)
                    first = False
                d = mk(sb1.at[inst, j], rb1.at[inst, j],
                       inst * 2 + j, s1_partner[inst])
                d.start()
                s1d[inst][j] = d

            d_qs = [dot_rows(KH[i] + (1 - qq[i]) * q_r, q_r, i)
                    for i in range(2)]
            d_qk = [dot_rows(KH[i] + qq[i] * q_r, q_r, i) for i in range(2)]

            s2d = [[None, None], [None, None]]
            for inst in range(2):
                s1d[inst][1 - qq[inst]].wait_recv()
                psum = (d_qs[inst]
                        + rb1[inst, 1 - qq[inst]].astype(jnp.float32))
                for sj in range(2):
                    sb2[inst, sj] = \
                        psum[sj * e_r:(sj + 1) * e_r, :].astype(jnp.bfloat16)
                    d = mk(sb2.at[inst, sj], rb2.at[inst, sj],
                           4 + inst * 2 + sj, s2_partner[inst])
                    d.start()
                    s2d[inst][sj] = d
            ksum = []
            for inst in range(2):
                s1d[inst][qq[inst]].wait_recv()
                ksum.append(d_qk[inst]
                            + rb1[inst, qq[inst]].astype(jnp.float32))

            s3d = [[None, None], [None, None]]
            s4d = [[None] * 4, [None] * 4]
            for sj in range(2):
                for inst in range(2):
                    s2d[inst][sj].wait_recv()
                    gv = _gelu(ksum[inst][sj * e_r:(sj + 1) * e_r, :]
                               + rb2[inst, sj].astype(jnp.float32))
                    sb3[inst, sj] = gv.astype(jnp.bfloat16)
                    d = mk(sb3.at[inst, sj], rb3.at[inst, sj],
                           8 + inst * 2 + sj, s2_partner[inst])
                    d.start()
                    s3d[inst][sj] = d
                    j4 = 2 * qq[inst] + sj
                    d = mk(sb3.at[inst, sj], rb4.at[inst, j4],
                           12 + inst * 4 + j4, s1_partner[inst])
                    d.start()
                    s4d[inst][j4] = d
                    r0 = KH[inst] + qq[inst] * q_r + sj * e_r
                    store_out(r0, inst, gv)

            for sj in range(2):
                for inst in range(2):
                    s3d[inst][sj].wait_recv()
                    j4 = 2 * (1 - qq[inst]) + sj
                    d = mk(rb3.at[inst, sj], rb4.at[inst, j4],
                           12 + inst * 4 + j4, s1_partner[inst])
                    d.start()
                    s4d[inst][j4] = d
                for inst in range(2):
                    r0 = KH[inst] + (1 - qq[inst]) * q_r + sj * e_r
                    store_out(r0, inst, rb3[inst, sj].astype(jnp.float32))

            qp = [qq[0], 1 - qq[1]]
            arrive = [[2 * qp[i], 2 * qp[i] + 1,
                       2 * (1 - qp[i]), 2 * (1 - qp[i]) + 1]
                      for i in range(2)]
            for k in range(4):
                for inst in range(2):
                    j4 = arrive[inst][k]
                    s4d[inst][j4].wait_recv()
                    r0 = SH[inst] + j4 * e_r
                    store_out(r0, inst, rb4[inst, j4].astype(jnp.float32))

            for inst in range(2):
                for j in range(2):
                    s1d[inst][j].wait_send()
                    s2d[inst][j].wait_send()
                    s3d[inst][j].wait_send()
                for j4 in range(4):
                    s4d[inst][j4].wait_send()
            for cp in odmas:
                cp.wait()

        for dev in range(N_DEV):
            @pl.when(my == dev)
            def _(dev=dev):
                run(dev)

    return pl.pallas_call(
        body,
        out_shape=jax.ShapeDtypeStruct((m, n), jnp.float32),
        in_specs=[
            pl.BlockSpec(memory_space=pl.ANY),
            pl.BlockSpec(memory_space=pl.ANY),
        ],
        out_specs=pl.BlockSpec(memory_space=pl.ANY),
        scratch_shapes=[
            pltpu.VMEM(x.shape, jnp.float32),
            pltpu.VMEM(w_mat.shape, jnp.float32),
            pltpu.VMEM((m, n), jnp.float32),
            pltpu.VMEM((2, 2, q_r, half_c), jnp.bfloat16),
            pltpu.VMEM((2, 2, q_r, half_c), jnp.bfloat16),
            pltpu.VMEM((2, 2, e_r, half_c), jnp.bfloat16),
            pltpu.VMEM((2, 2, e_r, half_c), jnp.bfloat16),
            pltpu.VMEM((2, 2, e_r, half_c), jnp.bfloat16),
            pltpu.VMEM((2, 2, e_r, half_c), jnp.bfloat16),
            pltpu.VMEM((2, 4, e_r, half_c), jnp.bfloat16),
            pltpu.SemaphoreType.DMA((20,)),
            pltpu.SemaphoreType.DMA((20,)),
            pltpu.SemaphoreType.DMA((16,)),
            pltpu.SemaphoreType.DMA((2,)),
        ],
        compiler_params=pltpu.CompilerParams(collective_id=0),
    )(x, w_mat)


# device time: 27396 ns/iter; 1.0054x vs baseline; 1.0054x over previous
import jax
import jax.numpy as jnp
from jax import lax
from jax.experimental import pallas as pl
from jax.experimental.pallas import tpu as pltpu

N_DEV = 4
GRAY = [0, 1, 1, 0]


def _gelu(y):
    c = 0.7978845608028654
    return 0.5 * y * (1.0 + jnp.tanh(c * (y + 0.044715 * (y * y * y))))


def kernel(x, w_mat):
    m, _ = x.shape
    _, n = w_mat.shape
    half_r = m // 2
    q_r = m // 4
    e_r = m // 8
    half_c = n // 2

    def body(x_ref, w_ref, out_ref,
             sb1, rb1, sb2, rb2, sb3, rb3, rb4,
             send_sems, recv_sems):
        my = lax.axis_index("i")
        p1t = lax.bitwise_xor(my, 1)
        p2t = 3 - my

        barrier = pltpu.get_barrier_semaphore()
        pl.semaphore_signal(barrier, inc=1, device_id=(p1t,),
                            device_id_type=pl.DeviceIdType.MESH)
        pl.semaphore_signal(barrier, inc=1, device_id=(p2t,),
                            device_id_type=pl.DeviceIdType.MESH)

        cols = [slice(0, half_c), slice(half_c, n)]
        wb = [w_ref[:, cols[i]].astype(jnp.bfloat16) for i in range(2)]

        def run(dev):
            p1, p2 = dev ^ 1, 3 - dev
            hh = [GRAY[dev], dev >> 1]
            qq = [dev >> 1, dev & 1]
            s1_partner = [p1, p2]
            s2_partner = [p2, p1]
            KH = [hh[i] * half_r for i in range(2)]
            SH = [(1 - hh[i]) * half_r for i in range(2)]

            def dot_rows(r0, nrows, inst):
                xb = x_ref[r0:r0 + nrows, :].astype(jnp.bfloat16)
                return jnp.dot(xb, wb[inst], preferred_element_type=jnp.float32)

            def mk(src, dst, idx, tgt):
                return pltpu.make_async_remote_copy(
                    src_ref=src, dst_ref=dst,
                    send_sem=send_sems.at[idx], recv_sem=recv_sems.at[idx],
                    device_id=(tgt,), device_id_type=pl.DeviceIdType.MESH)

            jf = [1 - qq[0], qq[1]]
            s1d = [[None, None], [None, None]]
            first = True
            for inst, j in [(0, jf[0]), (1, jf[1]),
                            (0, 1 - jf[0]), (1, 1 - jf[1])]:
                sb1[inst, j] = dot_rows(SH[inst] + j * q_r, q_r,
                                        inst).astype(jnp.bfloat16)
                if first:
                    pl.semaphore_wait(barrier, 2)
                    first = False
                d = mk(sb1.at[inst, j], rb1.at[inst, j],
                       inst * 2 + j, s1_partner[inst])
                d.start()
                s1d[inst][j] = d

            d_qs = [dot_rows(KH[i] + (1 - qq[i]) * q_r, q_r, i)
                    for i in range(2)]
            d_qk = [dot_rows(KH[i] + qq[i] * q_r, q_r, i) for i in range(2)]

            s2d = [[None, None], [None, None]]
            for inst in range(2):
                s1d[inst][1 - qq[inst]].wait_recv()
                psum = (d_qs[inst]
                        + rb1[inst, 1 - qq[inst]].astype(jnp.float32))
                for sj in range(2):
                    sb2[inst, sj] = \
                        psum[sj * e_r:(sj + 1) * e_r, :].astype(jnp.bfloat16)
                    d = mk(sb2.at[inst, sj], rb2.at[inst, sj],
                           4 + inst * 2 + sj, s2_partner[inst])
                    d.start()
                    s2d[inst][sj] = d
            ksum = []
            for inst in range(2):
                s1d[inst][qq[inst]].wait_recv()
                ksum.append(d_qk[inst]
                            + rb1[inst, qq[inst]].astype(jnp.float32))

            s3d = [[None, None], [None, None]]
            s4d = [[None] * 4, [None] * 4]
            for sj in range(2):
                for inst in range(2):
                    s2d[inst][sj].wait_recv()
                    gv = _gelu(ksum[inst][sj * e_r:(sj + 1) * e_r, :]
                               + rb2[inst, sj].astype(jnp.float32))
                    sb3[inst, sj] = gv.astype(jnp.bfloat16)
                    d = mk(sb3.at[inst, sj], rb3.at[inst, sj],
                           8 + inst * 2 + sj, s2_partner[inst])
                    d.start()
                    s3d[inst][sj] = d
                    j4 = 2 * qq[inst] + sj
                    d = mk(sb3.at[inst, sj], rb4.at[inst, j4],
                           12 + inst * 4 + j4, s1_partner[inst])
                    d.start()
                    s4d[inst][j4] = d
                    r0 = KH[inst] + qq[inst] * q_r + sj * e_r
                    out_ref[r0:r0 + e_r, cols[inst]] = gv

            for sj in range(2):
                for inst in range(2):
                    s3d[inst][sj].wait_recv()
                    j4 = 2 * (1 - qq[inst]) + sj
                    d = mk(rb3.at[inst, sj], rb4.at[inst, j4],
                           12 + inst * 4 + j4, s1_partner[inst])
                    d.start()
                    s4d[inst][j4] = d
                for inst in range(2):
                    r0 = KH[inst] + (1 - qq[inst]) * q_r + sj * e_r
                    out_ref[r0:r0 + e_r, cols[inst]] = \
                        rb3[inst, sj].astype(jnp.float32)

            qp = [qq[0], 1 - qq[1]]
            arrive = [[2 * qp[i], 2 * qp[i] + 1,
                       2 * (1 - qp[i]), 2 * (1 - qp[i]) + 1]
                      for i in range(2)]
            for k in range(4):
                for inst in range(2):
                    j4 = arrive[inst][k]
                    s4d[inst][j4].wait_recv()
                    r0 = SH[inst] + j4 * e_r
                    out_ref[r0:r0 + e_r, cols[inst]] = \
                        rb4[inst, j4].astype(jnp.float32)

            for inst in range(2):
                for j in range(2):
                    s1d[inst][j].wait_send()
                    s2d[inst][j].wait_send()
                    s3d[inst][j].wait_send()
                for j4 in range(4):
                    s4d[inst][j4].wait_send()

        for dev in range(N_DEV):
            @pl.when(my == dev)
            def _(dev=dev):
                run(dev)

    return pl.pallas_call(
        body,
        out_shape=jax.ShapeDtypeStruct((m, n), jnp.float32),
        in_specs=[
            pl.BlockSpec(memory_space=pltpu.VMEM),
            pl.BlockSpec(memory_space=pltpu.VMEM),
        ],
        out_specs=pl.BlockSpec(memory_space=pltpu.VMEM),
        scratch_shapes=[
            pltpu.VMEM((2, 2, q_r, half_c), jnp.bfloat16),
            pltpu.VMEM((2, 2, q_r, half_c), jnp.bfloat16),
            pltpu.VMEM((2, 2, e_r, half_c), jnp.bfloat16),
            pltpu.VMEM((2, 2, e_r, half_c), jnp.bfloat16),
            pltpu.VMEM((2, 2, e_r, half_c), jnp.bfloat16),
            pltpu.VMEM((2, 2, e_r, half_c), jnp.bfloat16),
            pltpu.VMEM((2, 4, e_r, half_c), jnp.bfloat16),
            pltpu.SemaphoreType.DMA((20,)),
            pltpu.SemaphoreType.DMA((20,)),
        ],
        compiler_params=pltpu.CompilerParams(collective_id=0),
    )(x, w_mat)
